# baseline (speedup 1.0000x reference)
import functools

import jax
import jax.numpy as jnp
from jax import lax
from jax.experimental import pallas as pl
from jax.experimental.pallas import tpu as pltpu
from jax.experimental.pallas import tpu_sc as plsc

B = 2
C = 64
H = 256
G = 64
N = G * G
EPS = 1e-6
BLK = 1024
WIN = BLK + 256
OFFSETS = (1, -1, G, -G)

NC = 2
NS = 16
L = 16
NW = NC * NS
ROWS_PER_W = (B * G) // NW
CC = 8


def _lane_gather(x, ind):
    dnums = lax.GatherDimensionNumbers(
        offset_dims=(), collapsed_slice_dims=(0,), start_index_map=(0,))
    return lax.gather(x, ind[:, None], dnums, (1,),
                      mode=lax.GatherScatterMode.PROMISE_IN_BOUNDS)


def _nodes_sc_body(xf_hbm, out_hbm, buf, dup, idx, sem_in, sem_out):
    wid = lax.axis_index("s") * NC + lax.axis_index("c")
    lane = lax.iota(jnp.int32, L)
    for m in range(ROWS_PER_W):
        t = wid * ROWS_PER_W + m
        b = t // G
        h = t % G
        acc = tuple(jnp.zeros((L,), jnp.float32) for _ in range(16))
        cps = [None, None]
        cps[0] = pltpu.async_copy(
            xf_hbm.at[b, pl.ds(0, CC), pl.ds(4 * h, 4), :], buf.at[0], sem_in)
        for cc in range(C // CC):
            cps[cc % 2].wait()
            if cc < C // CC - 1:
                cps[(cc + 1) % 2] = pltpu.async_copy(
                    xf_hbm.at[b, pl.ds(CC * (cc + 1), CC), pl.ds(4 * h, 4), :],
                    buf.at[(cc + 1) % 2], sem_in)

            def _chan(ch, a, _slot=cc % 2):
                for i in range(4):
                    a = tuple(
                        a[g] + buf[_slot, ch, i, pl.ds(16 * g, L)]
                        for g in range(16)
                    )
                return a

            acc = lax.fori_loop(0, CC, _chan, acc)

        half = 64 * (m % 2)
        for u in range(4):
            s = jnp.zeros((L,), jnp.float32)
            for p in range(4):
                g4 = acc[4 * u + p]
                g4 = g4 + _lane_gather(g4, lane ^ 1)
                g4 = g4 + _lane_gather(g4, lane ^ 2)
                comp = _lane_gather(g4, jnp.clip(4 * lane - 16 * p, 0, L - 1))
                sel = (lane >= 4 * p) & (lane < 4 * p + 4)
                s = jnp.where(sel, comp, s)
            s = s * (1.0 / 16.0)
            for ch in range(C):
                dup[ch, pl.ds(half + 16 * u, L)] = s

        if m % 2 == 1:
            h_even = h - 1
            base = b * (N // 2) + h_even // 2
            for u in range(4):
                idx[pl.ds(16 * u, L)] = base + (G // 2) * (16 * u + lane)
            pltpu.async_copy(dup, out_hbm.at[idx], sem_out).wait()


_nodes_sc = functools.partial(
    pl.kernel,
    out_type=jax.ShapeDtypeStruct((B * N // 2, 2 * C), jnp.float32),
    mesh=plsc.VectorSubcoreMesh(core_axis_name="c", subcore_axis_name="s"),
    scratch_types=[
        pltpu.VMEM((2, CC, 4, H), jnp.float32),
        pltpu.VMEM((C, 2 * C), jnp.float32),
        pltpu.VMEM((C,), jnp.int32),
        pltpu.SemaphoreType.DMA,
        pltpu.SemaphoreType.DMA,
    ],
)(_nodes_sc_body)


def _shift_up(a):
    return jnp.concatenate([a[:1], a[:-1]], axis=0)


def _shift_dn(a):
    return jnp.concatenate([a[1:], a[-1:]], axis=0)


def _shift_lf(a):
    return jnp.concatenate([a[:, :1], a[:, :-1]], axis=1)


def _shift_rt(a):
    return jnp.concatenate([a[:, 1:], a[:, -1:]], axis=1)


def _weights(xv_ref, w_ref):
    xv = xv_ref[0, 0]
    p1 = 0.125 * _shift_up(xv) + 0.75 * xv + 0.125 * _shift_dn(xv)
    p2 = 0.125 * _shift_lf(p1) + 0.75 * p1 + 0.125 * _shift_rt(p1)
    um = 1.0 - p2
    ri = lax.broadcasted_iota(jnp.int32, (G, G), 0)
    ci = lax.broadcasted_iota(jnp.int32, (G, G), 1)

    def t(x):
        return jnp.where(x > EPS, x, 0.0)

    w_ref[0, 0] = jnp.where(ci < G - 1, t(_shift_rt(um) - um), 0.0)
    w_ref[0, 1] = jnp.where(ci > 0, t(_shift_lf(um) - um), 0.0)
    w_ref[0, 2] = jnp.where(ri < G - 1, t(_shift_dn(um) - um), 0.0)
    w_ref[0, 3] = jnp.where(ri > 0, t(_shift_up(um) - um), 0.0)


def _fill(w_ref, adj_ref):
    k = pl.program_id(1)
    adj_ref[...] = jnp.zeros_like(adj_ref)
    start = pl.multiple_of(jnp.clip(BLK * k - 128, 0, N - WIN), 128)
    rowi = BLK * k + lax.broadcasted_iota(jnp.int32, (BLK, WIN), 0)
    coli = start + lax.broadcasted_iota(jnp.int32, (BLK, WIN), 1)
    delta = rowi - coli
    band = jnp.zeros((BLK, WIN), jnp.float32)
    for d, offs in enumerate(OFFSETS):
        wv = w_ref[0, d, pl.ds(start, WIN)]
        band = jnp.where(delta == offs, wv[None, :], band)
    adj_ref[0, :, pl.ds(start, WIN)] = band


def kernel(x_feat, x_var):
    w = pl.pallas_call(
        _weights,
        grid=(B,),
        in_specs=[pl.BlockSpec((1, 1, G, G), lambda b: (b, 0, 0, 0))],
        out_specs=pl.BlockSpec((1, 4, G, G), lambda b: (b, 0, 0, 0)),
        out_shape=jax.ShapeDtypeStruct((B, 4, G, G), jnp.float32),
    )(x_var)
    wf = w.reshape(B, 4, N)

    adjs = pl.pallas_call(
        _fill,
        grid=(B, N // BLK),
        in_specs=[pl.BlockSpec((1, 4, N), lambda b, k: (b, 0, 0))],
        out_specs=pl.BlockSpec((1, BLK, N), lambda b, k: (b, k, 0)),
        out_shape=jax.ShapeDtypeStruct((B, N, N), jnp.float32),
        compiler_params=pltpu.CompilerParams(
            dimension_semantics=("parallel", "arbitrary")),
    )(wf)
    nodes = _nodes_sc(x_feat).reshape(B, N, C)
    return nodes, adjs


# speedup vs baseline: 1.4227x; 1.4227x over previous
import jax
import jax.numpy as jnp
from jax import lax
from jax.experimental import pallas as pl
from jax.experimental.pallas import tpu as pltpu

B = 2
C = 64
H = 256
G = 64
N = G * G
EPS = 1e-6
C_CHUNK = 16
BLK = 1024
WIN = 1280
OFFSETS = (1, -1, G, -G)


def _shift_up(a):
    return jnp.concatenate([a[:1], a[:-1]], axis=0)


def _shift_dn(a):
    return jnp.concatenate([a[1:], a[-1:]], axis=0)


def _shift_lf(a):
    return jnp.concatenate([a[:, :1], a[:, :-1]], axis=1)


def _shift_rt(a):
    return jnp.concatenate([a[:, 1:], a[:, -1:]], axis=1)


def _weights(xv_ref, w_ref):
    xv = xv_ref[0, 0]
    p1 = 0.125 * _shift_up(xv) + 0.75 * xv + 0.125 * _shift_dn(xv)
    p2 = 0.125 * _shift_lf(p1) + 0.75 * p1 + 0.125 * _shift_rt(p1)
    um = 1.0 - p2
    ri = lax.broadcasted_iota(jnp.int32, (G, G), 0)
    ci = lax.broadcasted_iota(jnp.int32, (G, G), 1)

    def t(x):
        return jnp.where(x > EPS, x, 0.0)

    w_ref[0, 0] = jnp.where(ci < G - 1, t(_shift_rt(um) - um), 0.0)
    w_ref[0, 1] = jnp.where(ci > 0, t(_shift_lf(um) - um), 0.0)
    w_ref[0, 2] = jnp.where(ri < G - 1, t(_shift_dn(um) - um), 0.0)
    w_ref[0, 3] = jnp.where(ri > 0, t(_shift_up(um) - um), 0.0)


def _fused(xf_ref, w_ref, nodes_ref, adj_ref, acc_ref):
    k = pl.program_id(1)

    @pl.when(k == 0)
    def _init():
        acc_ref[...] = jnp.zeros_like(acc_ref)

    acc_ref[...] += jnp.sum(xf_ref[0], axis=0)

    adj_ref[...] = jnp.zeros_like(adj_ref)
    start = pl.multiple_of(jnp.clip(BLK * k - 128, 0, N - WIN), 128)
    rowi = BLK * k + lax.broadcasted_iota(jnp.int32, (BLK, WIN), 0)
    coli = start + lax.broadcasted_iota(jnp.int32, (BLK, WIN), 1)
    delta = rowi - coli
    band = jnp.zeros((BLK, WIN), jnp.float32)
    for d, offs in enumerate(OFFSETS):
        wv = w_ref[0, d, pl.ds(start, WIN)]
        band = jnp.where(delta == offs, wv[None, :], band)
    adj_ref[0, :, pl.ds(start, WIN)] = band

    @pl.when(k == pl.num_programs(1) - 1)
    def _fin():
        y = acc_ref[...]
        a = lax.broadcasted_iota(jnp.int32, (G, H), 0)
        b = lax.broadcasted_iota(jnp.int32, (G, H), 1) // 4
        P = jnp.where(a == b, 0.25, 0.0).astype(jnp.float32)
        s = jax.lax.dot_general(
            jax.lax.dot_general(P, y, (((1,), (0,)), ((), ())),
                                precision=lax.Precision.HIGHEST),
            P, (((1,), (1,)), ((), ())),
            precision=lax.Precision.HIGHEST)
        for t in range(G):
            nodes_ref[0, G * t:G * (t + 1), :] = s


def kernel(x_feat, x_var):
    w = pl.pallas_call(
        _weights,
        grid=(B,),
        in_specs=[pl.BlockSpec((1, 1, G, G), lambda b: (b, 0, 0, 0))],
        out_specs=pl.BlockSpec((1, 4, G, G), lambda b: (b, 0, 0, 0)),
        out_shape=jax.ShapeDtypeStruct((B, 4, G, G), jnp.float32),
    )(x_var)
    wf = w.reshape(B, 4, N)

    nodes, adjs = pl.pallas_call(
        _fused,
        grid=(B, N // BLK),
        in_specs=[
            pl.BlockSpec((1, C_CHUNK, H, H), lambda b, k: (b, k, 0, 0)),
            pl.BlockSpec((1, 4, N), lambda b, k: (b, 0, 0)),
        ],
        out_specs=[
            pl.BlockSpec((1, N, C), lambda b, k: (b, 0, 0)),
            pl.BlockSpec((1, BLK, N), lambda b, k: (b, k, 0)),
        ],
        out_shape=[
            jax.ShapeDtypeStruct((B, N, C), jnp.float32),
            jax.ShapeDtypeStruct((B, N, N), jnp.float32),
        ],
        scratch_shapes=[pltpu.VMEM((H, H), jnp.float32)],
        compiler_params=pltpu.CompilerParams(
            dimension_semantics=("parallel", "arbitrary")),
    )(x_feat, wf)
    return nodes, adjs


# speedup vs baseline: 1.4646x; 1.0295x over previous
import jax
import jax.numpy as jnp
from jax import lax
from jax.experimental import pallas as pl
from jax.experimental.pallas import tpu as pltpu

B = 2
C = 64
H = 256
G = 64
N = G * G
EPS = 1e-6
C_CHUNK = 16
BLK = 1024
WIN = 1280
OFFSETS = (1, -1, G, -G)


def _weights(xv_ref, w_ref):
    xv = xv_ref[0]
    ci = lax.broadcasted_iota(jnp.int32, (1, N), 1) % G
    first_col = ci == 0
    last_col = ci == G - 1

    def sh(a, off):
        if off < 0:
            return jnp.concatenate([a[:, :-off], a[:, :off]], axis=1)
        return jnp.concatenate([a[:, off:], a[:, -off:]], axis=1)

    up = sh(xv, -G)
    dn = sh(xv, G)
    p1 = 0.125 * up + 0.75 * xv + 0.125 * dn
    lf = jnp.where(first_col, p1, sh(p1, -1))
    rt = jnp.where(last_col, p1, sh(p1, 1))
    p2 = 0.125 * lf + 0.75 * p1 + 0.125 * rt
    um = 1.0 - p2

    def t(x):
        return jnp.where(x > EPS, x, 0.0)

    w_ref[0:1, :] = jnp.where(last_col, 0.0, t(sh(um, 1) - um))
    w_ref[1:2, :] = jnp.where(first_col, 0.0, t(sh(um, -1) - um))
    w_ref[2:3, :] = t(sh(um, G) - um)
    w_ref[3:4, :] = t(sh(um, -G) - um)


def _fused(xf_ref, xv_ref, nodes_ref, adj_ref, acc_ref, w_ref):
    k = pl.program_id(1)

    @pl.when(k == 0)
    def _init():
        acc_ref[...] = jnp.zeros_like(acc_ref)
        _weights(xv_ref, w_ref)

    acc_ref[...] += jnp.sum(xf_ref[0], axis=0)

    adj_ref[...] = jnp.zeros_like(adj_ref)
    start = pl.multiple_of(jnp.clip(BLK * k - 128, 0, N - WIN), 128)
    rowi = BLK * k + lax.broadcasted_iota(jnp.int32, (BLK, WIN), 0)
    coli = start + lax.broadcasted_iota(jnp.int32, (BLK, WIN), 1)
    delta = rowi - coli
    band = jnp.zeros((BLK, WIN), jnp.float32)
    for d, offs in enumerate(OFFSETS):
        wv = w_ref[d, pl.ds(start, WIN)]
        band = jnp.where(delta == offs, wv[None, :], band)
    adj_ref[0, :, pl.ds(start, WIN)] = band

    @pl.when(k == pl.num_programs(1) - 1)
    def _fin():
        y = acc_ref[...]
        a = lax.broadcasted_iota(jnp.int32, (G, H), 0)
        b = lax.broadcasted_iota(jnp.int32, (G, H), 1) // 4
        P = jnp.where(a == b, 0.25, 0.0).astype(jnp.float32)
        s = jax.lax.dot_general(
            jax.lax.dot_general(P, y, (((1,), (0,)), ((), ())),
                                precision=lax.Precision.HIGHEST),
            P, (((1,), (1,)), ((), ())),
            precision=lax.Precision.HIGHEST)
        for t in range(G):
            nodes_ref[0, G * t:G * (t + 1), :] = s


def kernel(x_feat, x_var):
    xv_flat = x_var.reshape(B, 1, N)
    nodes, adjs = pl.pallas_call(
        _fused,
        grid=(B, N // BLK),
        in_specs=[
            pl.BlockSpec((1, C_CHUNK, H, H), lambda b, k: (b, k, 0, 0)),
            pl.BlockSpec((1, 1, N), lambda b, k: (b, 0, 0)),
        ],
        out_specs=[
            pl.BlockSpec((1, N, C), lambda b, k: (b, 0, 0)),
            pl.BlockSpec((1, BLK, N), lambda b, k: (b, k, 0)),
        ],
        out_shape=[
            jax.ShapeDtypeStruct((B, N, C), jnp.float32),
            jax.ShapeDtypeStruct((B, N, N), jnp.float32),
        ],
        scratch_shapes=[
            pltpu.VMEM((H, H), jnp.float32),
            pltpu.VMEM((4, N), jnp.float32),
        ],
        compiler_params=pltpu.CompilerParams(
            dimension_semantics=("parallel", "arbitrary")),
    )(x_feat, xv_flat)
    return nodes, adjs
